# baseline (speedup 1.0000x reference)
import jax
import jax.numpy as jnp
from jax import lax
from jax.experimental import pallas as pl
from jax.experimental.pallas import tpu as pltpu
from jax.experimental.pallas import tpu_sc as plsc

N = 10000
E = 320000
IN_FEAT = 128
OUT_FEAT = 128
NUM_RELS = 32
NUM_BASES = 8

NC = 2
NS = 16
NW = NC * NS
CHUNK = 128
NBUF = 2
IBUF = 2 * NBUF
CPW = -(-E // (CHUNK * NW * IBUF)) * IBUF
NCHUNKS = CPW * NW
E_PAD = NCHUNKS * CHUNK
NPAD = 10240
ROWS_PER_TILE = NPAD // NS
BN = 1000


def _xw_body(w_comp_ref, weight_ref, x_ref, out_ref):
    r = pl.program_id(0)
    w = w_comp_ref[r, 0] * weight_ref[0]
    for b in range(1, NUM_BASES):
        w = w + w_comp_ref[r, b] * weight_ref[b]
    out_ref[0] = jnp.dot(x_ref[...], w, preferred_element_type=jnp.float32)


def _fetch_idd(idx_hbm, dst_hbm, wid, j, slot, sem):
    pltpu.async_copy(idx_hbm.at[wid, j], slot.at[0], sem)
    pltpu.async_copy(dst_hbm.at[wid, j], slot.at[1], sem)


def _wait_idd(idx_hbm, dst_hbm, wid, j, slot, sem):
    pltpu.make_async_copy(idx_hbm.at[wid, j], slot.at[0], sem).wait()
    pltpu.make_async_copy(dst_hbm.at[wid, j], slot.at[1], sem).wait()


def _sc_body(xw_hbm, idx_hbm, dst_hbm, zeros_hbm, out_hbm, *scratch):
    idd = scratch[:IBUF]
    rows = scratch[IBUF:IBUF + NBUF]
    h_shared = scratch[IBUF + NBUF]
    isem = scratch[IBUF + NBUF + 1:IBUF + NBUF + 1 + IBUF]
    gsem = scratch[IBUF + NBUF + 1 + IBUF:]
    c = lax.axis_index("c")
    s = lax.axis_index("s")
    wid = s * NC + c
    tile_base = s * ROWS_PER_TILE
    pltpu.sync_copy(zeros_hbm, h_shared.at[pl.ds(tile_base, ROWS_PER_TILE)])
    plsc.subcore_barrier()

    for j in range(IBUF):
        _fetch_idd(idx_hbm, dst_hbm, wid, j, idd[j], isem[j])
    for j in range(NBUF):
        _wait_idd(idx_hbm, dst_hbm, wid, j, idd[j], isem[j])
        pltpu.async_copy(xw_hbm.at[idd[j].at[0]], rows[j], gsem[j])

    def step(j0, carry):
        for u in range(IBUF):
            j = j0 * IBUF + u
            ib, rb = u, u % NBUF
            pltpu.make_async_copy(xw_hbm.at[idd[ib].at[0]], rows[rb],
                                  gsem[rb]).wait()
            pltpu.sync_copy(rows[rb], h_shared.at[idd[ib].at[1]], add=True)

            @pl.when(j + IBUF < CPW)
            def _():
                _fetch_idd(idx_hbm, dst_hbm, wid, j + IBUF, idd[ib], isem[ib])

            @pl.when(j + NBUF < CPW)
            def _():
                ib2 = (u + NBUF) % IBUF
                _wait_idd(idx_hbm, dst_hbm, wid, j + NBUF, idd[ib2], isem[ib2])
                pltpu.async_copy(xw_hbm.at[idd[ib2].at[0]], rows[rb],
                                 gsem[rb])
        return carry

    lax.fori_loop(0, CPW // IBUF, step, 0)
    plsc.subcore_barrier()
    pltpu.sync_copy(h_shared.at[pl.ds(tile_base, ROWS_PER_TILE)],
                    out_hbm.at[c, pl.ds(tile_base, ROWS_PER_TILE)])


def _prep_body(src_ref, dst_ref, ety_ref, idxo_ref, dsto_ref):
    er = E // CHUNK
    pr = (E_PAD - E) // CHUNK
    idxo_ref[:er] = ety_ref[...] * N + src_ref[...]
    dsto_ref[:er] = dst_ref[...]
    pad_e = (lax.broadcasted_iota(jnp.int32, (pr, CHUNK), 0) * CHUNK
             + lax.broadcasted_iota(jnp.int32, (pr, CHUNK), 1))
    idxo_ref[er:] = pad_e % 4096
    dsto_ref[er:] = N + pad_e % (NPAD - N)


def _selfloop_body(x_ref, lw_ref, bias_ref, out_ref):
    out_ref[...] = lax.dot_general(
        x_ref[...], lw_ref[...], (((1,), (1,)), ((), ())),
        preferred_element_type=jnp.float32) + bias_ref[0]


def _combine_body(h0_ref, parts_ref, out_ref):
    out_ref[...] = parts_ref[0] + parts_ref[1] + h0_ref[...]


def kernel(x, edge_index, etypes, weight, w_comp, h_bias, loop_weight):
    src2 = edge_index[0].astype(jnp.int32).reshape(E // CHUNK, CHUNK)
    dst2 = edge_index[1].astype(jnp.int32).reshape(E // CHUNK, CHUNK)
    ety2 = etypes.astype(jnp.int32).reshape(E // CHUNK, CHUNK)
    idx_p, dst_p = pl.pallas_call(
        _prep_body,
        out_shape=[jax.ShapeDtypeStruct((E_PAD // CHUNK, CHUNK), jnp.int32),
                   jax.ShapeDtypeStruct((E_PAD // CHUNK, CHUNK), jnp.int32)],
    )(src2, dst2, ety2)
    idx_p = idx_p.reshape(NW, CPW, CHUNK)
    dst_p = dst_p.reshape(NW, CPW, CHUNK)

    xw = pl.pallas_call(
        _xw_body,
        grid=(NUM_RELS,),
        in_specs=[
            pl.BlockSpec(memory_space=pltpu.SMEM),
            pl.BlockSpec((NUM_BASES, IN_FEAT, OUT_FEAT), lambda r: (0, 0, 0)),
            pl.BlockSpec((N, IN_FEAT), lambda r: (0, 0)),
        ],
        out_specs=pl.BlockSpec((1, N, OUT_FEAT), lambda r: (r, 0, 0)),
        out_shape=jax.ShapeDtypeStruct((NUM_RELS, N, OUT_FEAT), jnp.float32),
    )(w_comp, weight, x)
    xw_flat = xw.reshape(NUM_RELS * N, OUT_FEAT)

    zeros_rows = jnp.zeros((ROWS_PER_TILE, OUT_FEAT), jnp.float32)

    mesh = plsc.VectorSubcoreMesh(core_axis_name="c", subcore_axis_name="s",
                                  num_cores=NC, num_subcores=NS)
    parts = pl.kernel(
        _sc_body,
        out_type=jax.ShapeDtypeStruct((NC, NPAD, OUT_FEAT), jnp.float32),
        mesh=mesh,
        scratch_types=(
            [pltpu.VMEM((2, CHUNK), jnp.int32)] * IBUF
            + [pltpu.VMEM((CHUNK, OUT_FEAT), jnp.float32)] * NBUF
            + [pltpu.VMEM_SHARED((NPAD, OUT_FEAT), jnp.float32)]
            + [pltpu.SemaphoreType.DMA] * (IBUF + NBUF)
        ),
    )(xw_flat, idx_p, dst_p, zeros_rows)

    h0 = pl.pallas_call(
        _selfloop_body,
        grid=(N // BN,),
        in_specs=[
            pl.BlockSpec((BN, IN_FEAT), lambda nb: (nb, 0)),
            pl.BlockSpec((OUT_FEAT, IN_FEAT), lambda nb: (0, 0)),
            pl.BlockSpec((1, OUT_FEAT), lambda nb: (0, 0)),
        ],
        out_specs=pl.BlockSpec((BN, OUT_FEAT), lambda nb: (nb, 0)),
        out_shape=jax.ShapeDtypeStruct((N, OUT_FEAT), jnp.float32),
    )(x, loop_weight, h_bias.reshape(1, OUT_FEAT))

    h = pl.pallas_call(
        _combine_body,
        grid=(N // BN,),
        in_specs=[
            pl.BlockSpec((BN, OUT_FEAT), lambda nb: (nb, 0)),
            pl.BlockSpec((NC, BN, OUT_FEAT), lambda nb: (0, nb, 0)),
        ],
        out_specs=pl.BlockSpec((BN, OUT_FEAT), lambda nb: (nb, 0)),
        out_shape=jax.ShapeDtypeStruct((N, OUT_FEAT), jnp.float32),
    )(h0, parts)
    return h


# speedup vs baseline: 1.0882x; 1.0882x over previous
import jax
import jax.numpy as jnp
from jax import lax
from jax.experimental import pallas as pl
from jax.experimental.pallas import tpu as pltpu
from jax.experimental.pallas import tpu_sc as plsc

N = 10000
E = 320000
IN_FEAT = 128
OUT_FEAT = 128
NUM_RELS = 32
NUM_BASES = 8

NC = 2
NS = 16
NW = NC * NS
CHUNK = 128
NBUF = 2
IBUF = 2 * NBUF
CPW = -(-E // (CHUNK * NW * IBUF)) * IBUF
EPW = CPW * CHUNK
LAST_CHUNKS = E // CHUNK - (NW - 1) * CPW
NPAD = 10240
ROWS_PER_TILE = NPAD // NS
BN = 1000
VL = 16


def _xw_body(w_comp_ref, weight_ref, x_ref, out_ref):
    r = pl.program_id(0)
    w = w_comp_ref[r, 0] * weight_ref[0]
    for b in range(1, NUM_BASES):
        w = w + w_comp_ref[r, b] * weight_ref[b]
    out_ref[0] = jnp.dot(x_ref[...], w, preferred_element_type=jnp.float32)


def _fetch_edges(ei_hbm, ety_hbm, e0, se, idd, sem):
    pltpu.async_copy(ei_hbm.at[0, pl.ds(e0, CHUNK)], se.at[0], sem)
    pltpu.async_copy(ety_hbm.at[pl.ds(e0, CHUNK)], se.at[1], sem)
    pltpu.async_copy(ei_hbm.at[1, pl.ds(e0, CHUNK)], idd.at[1], sem)


def _wait_edges(ei_hbm, ety_hbm, e0, se, idd, sem):
    pltpu.make_async_copy(ei_hbm.at[0, pl.ds(e0, CHUNK)], se.at[0], sem).wait()
    pltpu.make_async_copy(ety_hbm.at[pl.ds(e0, CHUNK)], se.at[1], sem).wait()
    pltpu.make_async_copy(ei_hbm.at[1, pl.ds(e0, CHUNK)], idd.at[1],
                          sem).wait()


def _build_idx(se, idd):
    for k in range(CHUNK // VL):
        sl = pl.ds(k * VL, VL)
        idd[0, sl] = se[1, sl] * N + se[0, sl]


def _sc_body(xw_hbm, ei_hbm, ety_hbm, zeros_hbm, out_hbm, *scratch):
    se = scratch[:IBUF]
    idd = scratch[IBUF:2 * IBUF]
    rows = scratch[2 * IBUF:2 * IBUF + NBUF]
    h_shared = scratch[2 * IBUF + NBUF]
    isem = scratch[2 * IBUF + NBUF + 1:2 * IBUF + NBUF + 1 + IBUF]
    gsem = scratch[2 * IBUF + NBUF + 1 + IBUF:]
    c = lax.axis_index("c")
    s = lax.axis_index("s")
    wid = s * NC + c
    base = wid * EPW
    nchunks = jnp.where(wid == NW - 1, LAST_CHUNKS, CPW)
    tile_base = s * ROWS_PER_TILE
    pltpu.sync_copy(zeros_hbm, h_shared.at[pl.ds(tile_base, ROWS_PER_TILE)])
    plsc.subcore_barrier()

    for j in range(IBUF):
        _fetch_edges(ei_hbm, ety_hbm, base + j * CHUNK, se[j], idd[j], isem[j])
    for j in range(NBUF):
        _wait_edges(ei_hbm, ety_hbm, base + j * CHUNK, se[j], idd[j], isem[j])
        _build_idx(se[j], idd[j])
        pltpu.async_copy(xw_hbm.at[idd[j].at[0]], rows[j], gsem[j])

    def step(j0, carry):
        for u in range(IBUF):
            j = j0 * IBUF + u
            ib, rb = u, u % NBUF
            pltpu.make_async_copy(xw_hbm.at[idd[ib].at[0]], rows[rb],
                                  gsem[rb]).wait()
            pltpu.sync_copy(rows[rb], h_shared.at[idd[ib].at[1]], add=True)

            @pl.when(j + IBUF < nchunks)
            def _():
                _fetch_edges(ei_hbm, ety_hbm, base + (j + IBUF) * CHUNK,
                             se[ib], idd[ib], isem[ib])

            @pl.when(j + NBUF < nchunks)
            def _():
                ib2 = (u + NBUF) % IBUF
                _wait_edges(ei_hbm, ety_hbm, base + (j + NBUF) * CHUNK,
                            se[ib2], idd[ib2], isem[ib2])
                _build_idx(se[ib2], idd[ib2])
                pltpu.async_copy(xw_hbm.at[idd[ib2].at[0]], rows[rb],
                                 gsem[rb])
        return carry

    lax.fori_loop(0, nchunks // IBUF, step, 0)
    plsc.subcore_barrier()
    pltpu.sync_copy(h_shared.at[pl.ds(tile_base, ROWS_PER_TILE)],
                    out_hbm.at[c, pl.ds(tile_base, ROWS_PER_TILE)])


def _selfloop_body(x_ref, lw_ref, bias_ref, out_ref):
    out_ref[...] = lax.dot_general(
        x_ref[...], lw_ref[...], (((1,), (1,)), ((), ())),
        preferred_element_type=jnp.float32) + bias_ref[0]


def _combine_body(h0_ref, parts_ref, out_ref):
    out_ref[...] = parts_ref[0] + parts_ref[1] + h0_ref[...]


def kernel(x, edge_index, etypes, weight, w_comp, h_bias, loop_weight):
    edge_index = edge_index.astype(jnp.int32)
    etypes = etypes.astype(jnp.int32)

    xw = pl.pallas_call(
        _xw_body,
        grid=(NUM_RELS,),
        in_specs=[
            pl.BlockSpec(memory_space=pltpu.SMEM),
            pl.BlockSpec((NUM_BASES, IN_FEAT, OUT_FEAT), lambda r: (0, 0, 0)),
            pl.BlockSpec((N, IN_FEAT), lambda r: (0, 0)),
        ],
        out_specs=pl.BlockSpec((1, N, OUT_FEAT), lambda r: (r, 0, 0)),
        out_shape=jax.ShapeDtypeStruct((NUM_RELS, N, OUT_FEAT), jnp.float32),
    )(w_comp, weight, x)
    xw_flat = xw.reshape(NUM_RELS * N, OUT_FEAT)

    zeros_rows = jnp.zeros((ROWS_PER_TILE, OUT_FEAT), jnp.float32)

    mesh = plsc.VectorSubcoreMesh(core_axis_name="c", subcore_axis_name="s",
                                  num_cores=NC, num_subcores=NS)
    parts = pl.kernel(
        _sc_body,
        out_type=jax.ShapeDtypeStruct((NC, NPAD, OUT_FEAT), jnp.float32),
        mesh=mesh,
        scratch_types=(
            [pltpu.VMEM((2, CHUNK), jnp.int32)] * (2 * IBUF)
            + [pltpu.VMEM((CHUNK, OUT_FEAT), jnp.float32)] * NBUF
            + [pltpu.VMEM_SHARED((NPAD, OUT_FEAT), jnp.float32)]
            + [pltpu.SemaphoreType.DMA] * (IBUF + NBUF)
        ),
    )(xw_flat, edge_index, etypes, zeros_rows)

    h0 = pl.pallas_call(
        _selfloop_body,
        grid=(N // BN,),
        in_specs=[
            pl.BlockSpec((BN, IN_FEAT), lambda nb: (nb, 0)),
            pl.BlockSpec((OUT_FEAT, IN_FEAT), lambda nb: (0, 0)),
            pl.BlockSpec((1, OUT_FEAT), lambda nb: (0, 0)),
        ],
        out_specs=pl.BlockSpec((BN, OUT_FEAT), lambda nb: (nb, 0)),
        out_shape=jax.ShapeDtypeStruct((N, OUT_FEAT), jnp.float32),
    )(x, loop_weight, h_bias.reshape(1, OUT_FEAT))

    h = pl.pallas_call(
        _combine_body,
        grid=(N // BN,),
        in_specs=[
            pl.BlockSpec((BN, OUT_FEAT), lambda nb: (nb, 0)),
            pl.BlockSpec((NC, BN, OUT_FEAT), lambda nb: (0, nb, 0)),
        ],
        out_specs=pl.BlockSpec((BN, OUT_FEAT), lambda nb: (nb, 0)),
        out_shape=jax.ShapeDtypeStruct((N, OUT_FEAT), jnp.float32),
    )(h0, parts)
    return h
